# baseline (speedup 1.0000x reference)
import jax
import jax.numpy as jnp
from jax import lax
from jax.experimental import pallas as pl
from jax.experimental.pallas import tpu as pltpu
from jax.experimental.pallas import tpu_sc as plsc

B, H, S, D, U = 8, 16, 2048, 128, 32
SLAB = S * D
PCHUNK = U * D
HALF = B * H * SLAB
CH = 16384
CHPS = SLAB // CH
NCK = 8 * CHPS


def _body(kc_hbm, vc_hbm, kn_hbm, vn_hbm, pos_hbm, out_hbm,
          pos_v, b0, b1, b2, b3, pb, sg0, sg1, sg2, sg3, ss0, ss1, ss2, ss3,
          sp):
    c = lax.axis_index("c")
    s = lax.axis_index("s")
    wid = s * 2 + c
    b = wid // 4
    q = wid % 4

    pltpu.sync_copy(pos_hbm, pos_v)
    pos_b = pos_v[b][0]

    for j in range(4):
        noff = (b * H + q * 4 + j) * PCHUNK
        pltpu.async_copy(kn_hbm.at[pl.ds(noff, PCHUNK)],
                         pb.at[pl.ds((2 * j) * PCHUNK, PCHUNK)], sp)
        pltpu.async_copy(vn_hbm.at[pl.ds(noff, PCHUNK)],
                         pb.at[pl.ds((2 * j + 1) * PCHUNK, PCHUNK)], sp)

    bufs = (b0, b1, b2, b3)
    sg = (sg0, sg1, sg2, sg3)
    ss = (ss0, ss1, ss2, ss3)
    srcs = ((kc_hbm, 0), (vc_hbm, HALF))

    def soff(local):
        return (b * H + q * 4 + local // CHPS) * SLAB + (local % CHPS) * CH

    def gather(src, local, k):
        pltpu.async_copy(src.at[pl.ds(soff(local), CH)], bufs[k], sg[k])

    def scatter(kvhalf, local, k):
        pltpu.async_copy(
            bufs[k], out_hbm.at[pl.ds(kvhalf + soff(local), CH)], ss[k])

    def wait_g(k):
        pltpu.make_async_copy(kc_hbm.at[pl.ds(0, CH)], bufs[k], sg[k]).wait()

    def wait_s(k):
        pltpu.make_async_copy(bufs[k], out_hbm.at[pl.ds(0, CH)], ss[k]).wait()

    def static_step(ch):
        kv, local = divmod(ch, NCK // 2)
        k = ch % 4
        wait_g(k)
        scatter(srcs[kv][1], local, k)
        gch = ch + 2
        if gch < NCK:
            kv2, local2 = divmod(gch, NCK // 2)
            if ch >= 2:
                wait_s(gch % 4)
            gather(srcs[kv2][0], local2, gch % 4)

    def make_body(kv):
        src, kvhalf = srcs[kv]

        def body(i, carry):
            for k in range(4):
                local = 4 * i + k - kv * (NCK // 2)
                wait_g(k)
                scatter(kvhalf, local, k)
                wait_s((k + 2) % 4)
                gather(src, local + 2, (k + 2) % 4)
            return carry

        return body

    gather(kc_hbm, 0, 0)
    gather(kc_hbm, 1, 1)
    for ch in range(4):
        static_step(ch)
    lax.fori_loop(1, 15, make_body(0), None)
    for ch in range(60, 68):
        static_step(ch)
    lax.fori_loop(17, 31, make_body(1), None)
    for ch in range(NCK - 4, NCK):
        static_step(ch)
    for k in range(4):
        wait_s(k)

    pltpu.make_async_copy(kn_hbm.at[pl.ds(0, 8 * PCHUNK)], pb, sp).wait()
    for j in range(4):
        doff = (b * H + q * 4 + j) * SLAB + pos_b * D
        pltpu.async_copy(pb.at[pl.ds((2 * j) * PCHUNK, PCHUNK)],
                         out_hbm.at[pl.ds(doff, PCHUNK)], sp)
        pltpu.async_copy(pb.at[pl.ds((2 * j + 1) * PCHUNK, PCHUNK)],
                         out_hbm.at[pl.ds(HALF + doff, PCHUNK)], sp)
    pltpu.make_async_copy(pb, kn_hbm.at[pl.ds(0, 8 * PCHUNK)], sp).wait()


def kernel(k_new, v_new, cache_seqlens, qcache_seqlens, k_cache_buf, v_cache_buf):
    pos = (cache_seqlens - qcache_seqlens).astype(jnp.int32)
    pos_by_batch = jnp.broadcast_to(pos[:, None], (B, 16))
    mesh = plsc.VectorSubcoreMesh(core_axis_name="c", subcore_axis_name="s")
    out_flat = pl.kernel(
        _body,
        out_type=jax.ShapeDtypeStruct((2 * HALF,), jnp.float32),
        mesh=mesh,
        scratch_types=[
            pltpu.VMEM((B, 16), jnp.int32),
            pltpu.VMEM((CH,), jnp.float32),
            pltpu.VMEM((CH,), jnp.float32),
            pltpu.VMEM((CH,), jnp.float32),
            pltpu.VMEM((CH,), jnp.float32),
            pltpu.VMEM((8 * PCHUNK,), jnp.float32),
            pltpu.SemaphoreType.DMA,
            pltpu.SemaphoreType.DMA,
            pltpu.SemaphoreType.DMA,
            pltpu.SemaphoreType.DMA,
            pltpu.SemaphoreType.DMA,
            pltpu.SemaphoreType.DMA,
            pltpu.SemaphoreType.DMA,
            pltpu.SemaphoreType.DMA,
            pltpu.SemaphoreType.DMA,
        ],
    )(
        k_cache_buf.reshape(-1),
        v_cache_buf.reshape(-1),
        k_new.reshape(-1),
        v_new.reshape(-1),
        pos_by_batch,
    )
    return out_flat.reshape(2, B, H, S, D)
